# initial kernel scaffold (unmeasured)
import jax
import jax.numpy as jnp
from jax import lax
from jax.experimental import pallas as pl
from jax.experimental.pallas import tpu as pltpu

N_DEV = 4


def kernel(x, w_mat, scale_x, scale_w):
    m_glob, k_per = x.shape
    k_glob, n = w_mat.shape
    m_per = m_glob // N_DEV

    def body(x_ref, w_ref, sx_ref, sw_ref, out_ref,
             x8_ref, recv_ref, send_sems, recv_sems):
        my = lax.axis_index("i")

        x8_ref[:, :] = x_ref[:, :].astype(jnp.float8_e5m2)

        barrier = pltpu.get_barrier_semaphore()
        for off in (1, 2, 3):
            pl.semaphore_signal(
                barrier, inc=1,
                device_id=((my + off) % N_DEV,),
                device_id_type=pl.DeviceIdType.MESH,
            )
        pl.semaphore_wait(barrier, N_DEV - 1)

        rdmas = []
        for off in (1, 2, 3):
            dst = (my + off) % N_DEV
            rdma = pltpu.make_async_remote_copy(
                src_ref=x8_ref.at[pl.ds(dst * m_per, m_per)],
                dst_ref=recv_ref.at[off - 1],
                send_sem=send_sems.at[off - 1],
                recv_sem=recv_sems.at[off - 1],
                device_id=(dst,),
                device_id_type=pl.DeviceIdType.MESH,
            )
            rdma.start()
            rdmas.append(rdma)

        scale = sx_ref[0] * sw_ref[0]

        def w8_slice(src):
            return pl.load(
                w_ref, (pl.ds(src * k_per, k_per), slice(None))
            ).astype(jnp.float8_e5m2)

        own = x8_ref[pl.ds(my * m_per, m_per), :]
        acc = jnp.dot(own, w8_slice(my), preferred_element_type=jnp.float32)

        for off in (1, 3, 2):
            rdmas[off - 1].wait_recv()
            src = (my - off) % N_DEV
            acc = acc + jnp.dot(recv_ref[off - 1], w8_slice(src),
                                preferred_element_type=jnp.float32)

        out_ref[:, :] = jnp.maximum(acc * scale, 0.0)

        for off in (1, 2, 3):
            rdmas[off - 1].wait_send()

    return pl.pallas_call(
        body,
        out_shape=jax.ShapeDtypeStruct((m_per, n), jnp.float32),
        in_specs=[
            pl.BlockSpec(memory_space=pltpu.VMEM),
            pl.BlockSpec(memory_space=pltpu.VMEM),
            pl.BlockSpec(memory_space=pltpu.SMEM),
            pl.BlockSpec(memory_space=pltpu.SMEM),
        ],
        out_specs=pl.BlockSpec(memory_space=pltpu.VMEM),
        scratch_shapes=[
            pltpu.VMEM((m_glob, k_per), jnp.float8_e5m2),
            pltpu.VMEM((N_DEV - 1, m_per, k_per), jnp.float8_e5m2),
            pltpu.SemaphoreType.DMA((N_DEV - 1,)),
            pltpu.SemaphoreType.DMA((N_DEV - 1,)),
        ],
        compiler_params=pltpu.CompilerParams(collective_id=0),
    )(x, w_mat, scale_x, scale_w)


# baseline (device time: 43193 ns/iter reference)
import jax
import jax.numpy as jnp
from jax import lax
from jax.experimental import pallas as pl
from jax.experimental.pallas import tpu as pltpu

N_DEV = 4


def kernel(x, w_mat, scale_x, scale_w):
    m_glob, k_per = x.shape
    k_glob, n = w_mat.shape
    m_per = m_glob // N_DEV

    xseq = (1, 3, 2, 0)
    wseq = (0, 1, 3, 2)

    def body(x_hbm, w_hbm, sx_ref, sw_ref, out_ref,
             x8_ref, recv_ref, xstage, wstage,
             xsems, wsems, send_sems, recv_sems):
        my = lax.axis_index("i")

        barrier = pltpu.get_barrier_semaphore()
        for off in (1, 2, 3):
            pl.semaphore_signal(
                barrier, inc=1,
                device_id=((my + off) % N_DEV,),
                device_id_type=pl.DeviceIdType.MESH,
            )
        pl.semaphore_wait(barrier, N_DEV - 1)

        def xcopy(i):
            dst = (my + xseq[i]) % N_DEV
            return pltpu.make_async_copy(
                x_hbm.at[pl.ds(dst * m_per, m_per)],
                xstage.at[i % 2],
                xsems.at[i % 2],
            )

        rdmas = {}
        xcopy(0).start()
        for i, off in enumerate(xseq):
            if i + 1 < N_DEV:
                xcopy(i + 1).start()
            xcopy(i).wait()
            slot = (off - 1) if off else 3
            x8_ref[slot] = xstage[i % 2].astype(jnp.float8_e5m2)
            if off:
                rdma = pltpu.make_async_remote_copy(
                    src_ref=x8_ref.at[slot],
                    dst_ref=recv_ref.at[slot],
                    send_sem=send_sems.at[slot],
                    recv_sem=recv_sems.at[slot],
                    device_id=((my + off) % N_DEV,),
                    device_id_type=pl.DeviceIdType.MESH,
                )
                rdma.start()
                rdmas[off] = rdma

        def wcopy(i):
            src = (my - wseq[i]) % N_DEV
            return pltpu.make_async_copy(
                w_hbm.at[pl.ds(src * k_per, k_per)],
                wstage.at[i % 2],
                wsems.at[i % 2],
            )

        scale = sx_ref[0] * sw_ref[0]

        wcopy(0).start()
        for i, off in enumerate(wseq):
            if i + 1 < N_DEV:
                wcopy(i + 1).start()
            wcopy(i).wait()
            w8 = wstage[i % 2].astype(jnp.float8_e5m2)
            if off:
                rdmas[off].wait_recv()
                a = recv_ref[off - 1]
            else:
                a = x8_ref[3]
            contrib = jnp.dot(a, w8, preferred_element_type=jnp.float32)
            if i == 0:
                out_ref[:, :] = contrib
            elif i < N_DEV - 1:
                out_ref[:, :] = out_ref[:, :] + contrib
            else:
                out_ref[:, :] = jnp.maximum(
                    (out_ref[:, :] + contrib) * scale, 0.0
                )

        for off in (1, 2, 3):
            rdmas[off].wait_send()

    return pl.pallas_call(
        body,
        out_shape=jax.ShapeDtypeStruct((m_per, n), jnp.float32),
        in_specs=[
            pl.BlockSpec(memory_space=pltpu.MemorySpace.HBM),
            pl.BlockSpec(memory_space=pltpu.MemorySpace.HBM),
            pl.BlockSpec(memory_space=pltpu.SMEM),
            pl.BlockSpec(memory_space=pltpu.SMEM),
        ],
        out_specs=pl.BlockSpec(memory_space=pltpu.VMEM),
        scratch_shapes=[
            pltpu.VMEM((N_DEV, m_per, k_per), jnp.float8_e5m2),
            pltpu.VMEM((N_DEV - 1, m_per, k_per), jnp.float8_e5m2),
            pltpu.VMEM((2, m_per, k_per), jnp.float32),
            pltpu.VMEM((2, k_per, n), jnp.float32),
            pltpu.SemaphoreType.DMA((2,)),
            pltpu.SemaphoreType.DMA((2,)),
            pltpu.SemaphoreType.DMA((N_DEV - 1,)),
            pltpu.SemaphoreType.DMA((N_DEV - 1,)),
        ],
        compiler_params=pltpu.CompilerParams(
            collective_id=0,
            vmem_limit_bytes=52 * 1024 * 1024,
        ),
    )(x, w_mat, scale_x, scale_w)
